# baseline (speedup 1.0000x reference)
import jax
import jax.numpy as jnp
from jax import lax
from jax.experimental import pallas as pl
from jax.experimental.pallas import tpu as pltpu
from jax.experimental.pallas import tpu_sc as plsc

NUM_NODES = 100000
NUM_RELS = 64
H_DIM = 128

N_HN = 100000
N_HE = 320000

NC = 2
NS = 16
NW = NC * NS

CHUNK = 128

N_CHUNKS_N = 25
N_CHUNKS_E = 79
N_PAD = NW * N_CHUNKS_N * CHUNK
E_PAD = NW * N_CHUNKS_E * CHUNK


def _emb_kernel(hn_hbm, he_hbm, n_table_hbm, e_table_hbm,
                n_out_hbm, e_out_hbm,
                idx_buf, rows_buf, sem):
    wid = lax.axis_index("s") * NC + lax.axis_index("c")

    def run_table(idx_hbm, table_hbm, out_hbm, n_chunks):
        base = wid * (n_chunks * CHUNK)

        def body(j, _):
            off = base + j * CHUNK
            pltpu.sync_copy(idx_hbm.at[pl.ds(off, CHUNK)], idx_buf)
            pltpu.async_copy(table_hbm.at[idx_buf], rows_buf, sem).wait()
            pltpu.sync_copy(rows_buf, out_hbm.at[pl.ds(off, CHUNK)])
            return 0

        lax.fori_loop(0, n_chunks, body, 0)

    run_table(hn_hbm, n_table_hbm, n_out_hbm, N_CHUNKS_N)
    run_table(he_hbm, e_table_hbm, e_out_hbm, N_CHUNKS_E)


@jax.jit
def _run(hn_pad, he_pad, n_table, e_table):
    mesh = plsc.VectorSubcoreMesh(core_axis_name="c", subcore_axis_name="s")
    f = pl.kernel(
        _emb_kernel,
        out_type=(
            jax.ShapeDtypeStruct((N_PAD, H_DIM), jnp.float32),
            jax.ShapeDtypeStruct((E_PAD, H_DIM), jnp.float32),
        ),
        mesh=mesh,
        scratch_types=[
            pltpu.VMEM((CHUNK,), jnp.int32),
            pltpu.VMEM((CHUNK, H_DIM), jnp.float32),
            pltpu.SemaphoreType.DMA,
        ],
    )
    return f(hn_pad, he_pad, n_table, e_table)


def kernel(g, hn, r, he, norm, n_table, e_table):
    hn_flat = hn.reshape(-1).astype(jnp.int32)
    he_flat = he.reshape(-1).astype(jnp.int32)
    hn_pad = jnp.pad(hn_flat, (0, N_PAD - N_HN))
    he_pad = jnp.pad(he_flat, (0, E_PAD - N_HE))
    n_full, e_full = _run(hn_pad, he_pad, n_table, e_table)
    return (n_full[:N_HN], e_full[:N_HE])
